# Initial kernel scaffold; baseline (speedup 1.0000x reference)
import jax
import jax.numpy as jnp
from jax.experimental import pallas as pl


def kernel(x, edge_index, batch, target, params):
    raise NotImplementedError("write your pallas kernel here")


# speedup vs baseline: 17.6237x; 17.6237x over previous
import functools

import jax
import jax.numpy as jnp
from jax import lax
from jax.experimental import pallas as pl
from jax.experimental.pallas import tpu as pltpu
from jax.experimental.pallas import tpu_sc as plsc

N = 10000
NP = 10240
E = 320000
WIN = 128
EW = 80
NWK = 32
EPW = EW * WIN
EP = NWK * EPW
G = 64
D = 128
BLK = 512
NBLK = NP // BLK
NF = 32
LPROT = 1024
LCONV = LPROT - 8 + 1
ROWS_PER_SUB = NP // 16

_SC_MESH = dict(core_axis_name="c", subcore_axis_name="s", num_cores=2,
                num_subcores=16)


def _sc_feat_body(y_hbm, src_hbm, dst2_hbm, zero_hbm, out_hbm,
                  src_v, dstw_v, rows_v, acc_sh, sem):
    cid = lax.axis_index("c")
    sid = lax.axis_index("s")
    wid = cid * 16 + sid
    pltpu.sync_copy(zero_hbm.at[pl.ds(sid * ROWS_PER_SUB, ROWS_PER_SUB)],
                    acc_sh.at[pl.ds(sid * ROWS_PER_SUB, ROWS_PER_SUB)])
    pltpu.sync_copy(src_hbm.at[pl.ds(wid * EPW, EPW)], src_v)
    pltpu.sync_copy(dst2_hbm.at[pl.ds(wid * EW, EW)], dstw_v)
    plsc.subcore_barrier()

    @pl.loop(0, EW)
    def _win(j):
        pltpu.async_copy(y_hbm.at[src_v.at[pl.ds(j * WIN, WIN)]], rows_v,
                         sem).wait()
        pltpu.sync_copy(rows_v, acc_sh.at[dstw_v.at[j]], add=True)

    plsc.subcore_barrier()
    pltpu.sync_copy(acc_sh.at[pl.ds(sid * ROWS_PER_SUB, ROWS_PER_SUB)],
                    out_hbm.at[cid, pl.ds(sid * ROWS_PER_SUB, ROWS_PER_SUB)])


def _sc_feat_agg(y, src_flat, dst2, zero2d):
    return pl.kernel(
        _sc_feat_body,
        out_type=jax.ShapeDtypeStruct((2, NP, D), jnp.float32),
        mesh=plsc.VectorSubcoreMesh(**_SC_MESH),
        scratch_types=[
            pltpu.VMEM((EPW,), jnp.int32),
            pltpu.VMEM((EW, WIN), jnp.int32),
            pltpu.VMEM((WIN, D), jnp.float32),
            pltpu.VMEM_SHARED((NP, D), jnp.float32),
            pltpu.SemaphoreType.DMA,
        ],
    )(y, src_flat, dst2, zero2d)


def _sc_scal_body(z_hbm, src_hbm, dst2_hbm, zero_hbm, out_hbm,
                  src_v, dstw_v, vals_v, acc_sh, sem):
    cid = lax.axis_index("c")
    sid = lax.axis_index("s")
    wid = cid * 16 + sid
    pltpu.sync_copy(zero_hbm.at[pl.ds(sid * ROWS_PER_SUB, ROWS_PER_SUB)],
                    acc_sh.at[pl.ds(sid * ROWS_PER_SUB, ROWS_PER_SUB)])
    pltpu.sync_copy(src_hbm.at[pl.ds(wid * EPW, EPW)], src_v)
    pltpu.sync_copy(dst2_hbm.at[pl.ds(wid * EW, EW)], dstw_v)
    plsc.subcore_barrier()

    @pl.loop(0, EW)
    def _win(j):
        pltpu.async_copy(z_hbm.at[src_v.at[pl.ds(j * WIN, WIN)]], vals_v,
                         sem).wait()
        pltpu.sync_copy(vals_v, acc_sh.at[dstw_v.at[j]], add=True)

    plsc.subcore_barrier()
    pltpu.sync_copy(acc_sh.at[pl.ds(sid * ROWS_PER_SUB, ROWS_PER_SUB)],
                    out_hbm.at[cid, pl.ds(sid * ROWS_PER_SUB, ROWS_PER_SUB)])


def _sc_scal_agg(z, src_flat, dst2, zero1d):
    return pl.kernel(
        _sc_scal_body,
        out_type=jax.ShapeDtypeStruct((2, NP), jnp.float32),
        mesh=plsc.VectorSubcoreMesh(**_SC_MESH),
        scratch_types=[
            pltpu.VMEM((EPW,), jnp.int32),
            pltpu.VMEM((EW, WIN), jnp.int32),
            pltpu.VMEM((WIN,), jnp.float32),
            pltpu.VMEM_SHARED((NP,), jnp.float32),
            pltpu.SemaphoreType.DMA,
        ],
    )(z, src_flat, dst2, zero1d)


def _sc_deg_body(dst2_hbm, zero_hbm, out_hbm, dstw_v, vals_v, acc_sh):
    cid = lax.axis_index("c")
    sid = lax.axis_index("s")
    wid = cid * 16 + sid
    pltpu.sync_copy(zero_hbm.at[pl.ds(sid * ROWS_PER_SUB, ROWS_PER_SUB)],
                    acc_sh.at[pl.ds(sid * ROWS_PER_SUB, ROWS_PER_SUB)])
    for i in range(WIN // 16):
        vals_v[pl.ds(i * 16, 16)] = jnp.ones((16,), jnp.float32)
    pltpu.sync_copy(dst2_hbm.at[pl.ds(wid * EW, EW)], dstw_v)
    plsc.subcore_barrier()

    @pl.loop(0, EW)
    def _win(j):
        pltpu.sync_copy(vals_v, acc_sh.at[dstw_v.at[j]], add=True)

    plsc.subcore_barrier()
    pltpu.sync_copy(acc_sh.at[pl.ds(sid * ROWS_PER_SUB, ROWS_PER_SUB)],
                    out_hbm.at[cid, pl.ds(sid * ROWS_PER_SUB, ROWS_PER_SUB)])


def _sc_deg(dst2, zero1d):
    return pl.kernel(
        _sc_deg_body,
        out_type=jax.ShapeDtypeStruct((2, NP), jnp.float32),
        mesh=plsc.VectorSubcoreMesh(**_SC_MESH),
        scratch_types=[
            pltpu.VMEM((EW, WIN), jnp.int32),
            pltpu.VMEM((WIN,), jnp.float32),
            pltpu.VMEM_SHARED((NP,), jnp.float32),
        ],
    )(dst2, zero1d)


def _pre_body(degp_ref, x_ref, w_ref, dinv_ref, y_ref):
    i = pl.program_id(0)
    degrow = degp_ref[0:1, :] + degp_ref[1:2, :] + 1.0
    deg = jnp.transpose(degrow)
    rowid = lax.broadcasted_iota(jnp.int32, (BLK, 1), 0) + i * BLK
    dinv = jnp.where(rowid < N, lax.rsqrt(deg), 0.0)
    dinv_ref[...] = dinv
    y_ref[...] = dinv * jnp.dot(x_ref[...], w_ref[...],
                                preferred_element_type=jnp.float32)


def _pre(degp, xp, w):
    return pl.pallas_call(
        _pre_body,
        grid=(NBLK,),
        in_specs=[
            pl.BlockSpec((2, BLK), lambda i: (0, i)),
            pl.BlockSpec((BLK, D), lambda i: (i, 0)),
            pl.BlockSpec((D, D), lambda i: (0, 0)),
        ],
        out_specs=[
            pl.BlockSpec((BLK, 1), lambda i: (i, 0)),
            pl.BlockSpec((BLK, D), lambda i: (i, 0)),
        ],
        out_shape=[
            jax.ShapeDtypeStruct((NP, 1), jnp.float32),
            jax.ShapeDtypeStruct((NP, D), jnp.float32),
        ],
    )(degp, xp, w)


def _p1_body(parts_ref, y_ref, dinv_ref, b_ref, t_ref, stats_ref):
    i = pl.program_id(0)
    dinv = dinv_ref[...]
    t = dinv * (parts_ref[0] + parts_ref[1] + y_ref[...]) + b_ref[...]
    t_ref[...] = t
    mask = (dinv > 0.0).astype(jnp.float32)
    tm = t * mask

    @pl.when(i == 0)
    def _():
        stats_ref[...] = jnp.zeros_like(stats_ref)

    stats_ref[0:1, :] += jnp.sum(tm, axis=0, keepdims=True)


def _p1(parts, y, dinv, b):
    return pl.pallas_call(
        _p1_body,
        grid=(NBLK,),
        in_specs=[
            pl.BlockSpec((2, BLK, D), lambda i: (0, i, 0)),
            pl.BlockSpec((BLK, D), lambda i: (i, 0)),
            pl.BlockSpec((BLK, 1), lambda i: (i, 0)),
            pl.BlockSpec((1, D), lambda i: (0, 0)),
        ],
        out_specs=[
            pl.BlockSpec((BLK, D), lambda i: (i, 0)),
            pl.BlockSpec((8, D), lambda i: (0, 0)),
        ],
        out_shape=[
            jax.ShapeDtypeStruct((NP, D), jnp.float32),
            jax.ShapeDtypeStruct((8, D), jnp.float32),
        ],
    )(parts, y, dinv, b.reshape(1, D))


def _p1b_body(t_ref, stats_ref, dinv_ref, ssq_ref):
    i = pl.program_id(0)
    mean = stats_ref[0:1, :] * (1.0 / N)
    mask = (dinv_ref[...] > 0.0).astype(jnp.float32)
    dv = (t_ref[...] - mean) * mask

    @pl.when(i == 0)
    def _():
        ssq_ref[...] = jnp.zeros_like(ssq_ref)

    ssq_ref[0:1, :] += jnp.sum(dv * dv, axis=0, keepdims=True)


def _p1b(t, stats, dinv):
    return pl.pallas_call(
        _p1b_body,
        grid=(NBLK,),
        in_specs=[
            pl.BlockSpec((BLK, D), lambda i: (i, 0)),
            pl.BlockSpec((8, D), lambda i: (0, 0)),
            pl.BlockSpec((BLK, 1), lambda i: (i, 0)),
        ],
        out_specs=pl.BlockSpec((8, D), lambda i: (0, 0)),
        out_shape=jax.ShapeDtypeStruct((8, D), jnp.float32),
    )(t, stats, dinv)


def _p2_body(t_ref, stats_ref, ssq_ref, g_ref, be_ref, wp_ref, dinv_ref,
             h_ref, z_ref):
    mean = stats_ref[0:1, :] * (1.0 / N)
    var = ssq_ref[0:1, :] * (1.0 / N)
    h = jnp.maximum(
        (t_ref[...] - mean) * lax.rsqrt(var + 1e-5) * g_ref[...] + be_ref[...],
        0.0)
    h_ref[...] = h
    z = jnp.dot(h, wp_ref[...], preferred_element_type=jnp.float32)[:, 0:1]
    z_ref[...] = dinv_ref[...] * z


def _p2(t, stats, ssq, g, be, wp_row, dinv):
    return pl.pallas_call(
        _p2_body,
        grid=(NBLK,),
        in_specs=[
            pl.BlockSpec((BLK, D), lambda i: (i, 0)),
            pl.BlockSpec((8, D), lambda i: (0, 0)),
            pl.BlockSpec((8, D), lambda i: (0, 0)),
            pl.BlockSpec((1, D), lambda i: (0, 0)),
            pl.BlockSpec((1, D), lambda i: (0, 0)),
            pl.BlockSpec((D, D), lambda i: (0, 0)),
            pl.BlockSpec((BLK, 1), lambda i: (i, 0)),
        ],
        out_specs=[
            pl.BlockSpec((BLK, D), lambda i: (i, 0)),
            pl.BlockSpec((BLK, 1), lambda i: (i, 0)),
        ],
        out_shape=[
            jax.ShapeDtypeStruct((NP, D), jnp.float32),
            jax.ShapeDtypeStruct((NP, 1), jnp.float32),
        ],
    )(t, stats, ssq, g.reshape(1, D), be.reshape(1, D), wp_row, dinv)


def _p3_body(gmin_ref, gmax_ref, h_ref, z_ref, saggp_ref, dinv_ref, bp_ref,
             batch_ref, w_ref, rmax_ref, rsum_ref, cnt_ref, y_ref=None,
             *, with_y):
    i = pl.program_id(0)
    sagg = jnp.transpose(saggp_ref[0:1, :] + saggp_ref[1:2, :])
    s = dinv_ref[...] * (sagg + z_ref[...]) + bp_ref[...]
    gt = h_ref[...] * jnp.tanh(s)

    bvec = batch_ref[0]
    gids = lax.broadcasted_iota(jnp.int32, (G, 1), 0)
    onehot = (bvec == gids).astype(jnp.float32)

    @pl.when(i == 0)
    def _():
        rsum_ref[...] = jnp.zeros_like(rsum_ref)
        cnt_ref[...] = jnp.zeros_like(cnt_ref)
        rmax_ref[...] = jnp.full_like(rmax_ref, -1e30)

    rsum_ref[...] += jnp.dot(onehot, gt, preferred_element_type=jnp.float32,
                             precision=lax.Precision.HIGHEST)
    cnt_ref[...] += jnp.sum(onehot, axis=1, keepdims=True)

    bcol = jnp.transpose(bvec)
    gmin = gmin_ref[i]
    gmax = gmax_ref[i]
    for gg in range(G):
        @pl.when((gmin <= gg) & (gg <= gmax))
        def _():
            m = bcol == gg
            colmax = jnp.max(jnp.where(m, gt, -1e30), axis=0, keepdims=True)
            rmax_ref[gg:gg + 1, :] = jnp.maximum(rmax_ref[gg:gg + 1, :],
                                                 colmax)
    if with_y:
        y_ref[...] = dinv_ref[...] * jnp.dot(
            gt, w_ref[...], preferred_element_type=jnp.float32)


def _p3(gmin, gmax, h, z, saggp, dinv, bp, bat2, w_next):
    with_y = w_next is not None
    if not with_y:
        w_next = jnp.zeros((D, D), jnp.float32)
    out_specs = [
        pl.BlockSpec((G, D), lambda i: (0, 0)),
        pl.BlockSpec((G, D), lambda i: (0, 0)),
        pl.BlockSpec((G, 1), lambda i: (0, 0)),
    ]
    out_shape = [
        jax.ShapeDtypeStruct((G, D), jnp.float32),
        jax.ShapeDtypeStruct((G, D), jnp.float32),
        jax.ShapeDtypeStruct((G, 1), jnp.float32),
    ]
    if with_y:
        out_specs.append(pl.BlockSpec((BLK, D), lambda i: (i, 0)))
        out_shape.append(jax.ShapeDtypeStruct((NP, D), jnp.float32))
    return pl.pallas_call(
        functools.partial(_p3_body, with_y=with_y),
        grid=(NBLK,),
        in_specs=[
            pl.BlockSpec(memory_space=pltpu.SMEM),
            pl.BlockSpec(memory_space=pltpu.SMEM),
            pl.BlockSpec((BLK, D), lambda i: (i, 0)),
            pl.BlockSpec((BLK, 1), lambda i: (i, 0)),
            pl.BlockSpec((2, BLK), lambda i: (0, i)),
            pl.BlockSpec((BLK, 1), lambda i: (i, 0)),
            pl.BlockSpec((1, 1), lambda i: (0, 0)),
            pl.BlockSpec((1, 1, BLK), lambda i: (i, 0, 0)),
            pl.BlockSpec((D, D), lambda i: (0, 0)),
        ],
        out_specs=out_specs,
        out_shape=out_shape,
    )(gmin, gmax, h, z, saggp, dinv, bp.reshape(1, 1), bat2, w_next)


def _conv_body(wt_ref, tgt_ref, bc_ref, out_ref):
    p = jnp.dot(wt_ref[...], tgt_ref[0], preferred_element_type=jnp.float32)
    acc = bc_ref[...] + jnp.zeros((NF, LPROT), jnp.float32)
    acc = acc + p[0:NF, :]
    for k in range(1, 8):
        acc = acc + jnp.roll(p[k * NF:(k + 1) * NF, :], -k, axis=1)
    out_ref[0] = acc


def _conv(wt, target, bc):
    return pl.pallas_call(
        _conv_body,
        grid=(G,),
        in_specs=[
            pl.BlockSpec((8 * NF, 640), lambda b: (0, 0)),
            pl.BlockSpec((1, 640, LPROT), lambda b: (b, 0, 0)),
            pl.BlockSpec((NF, 1), lambda b: (0, 0)),
        ],
        out_specs=pl.BlockSpec((1, NF, LPROT), lambda b: (b, 0, 0)),
        out_shape=jax.ShapeDtypeStruct((G, NF, LPROT), jnp.float32),
    )(wt, target, bc)


def _bn64(v, g, be):
    mean = jnp.mean(v, axis=0, keepdims=True)
    dv = v - mean
    var = jnp.mean(dv * dv, axis=0, keepdims=True)
    return dv * lax.rsqrt(var + 1e-5) * g + be


def _head_body(rmax1, rmax2, rmax3, rsum1, rsum2, rsum3, cnt_ref,
               convf_ref, wfx_ref, bfx_ref, g6_ref, be6_ref,
               wg1_ref, bg1_ref, g4_ref, be4_ref,
               wf1_ref, bf1_ref, g7_ref, be7_ref,
               wf2_ref, bf2_ref, g8_ref, be8_ref,
               wo_ref, bo_ref, out_ref):
    cnt = jnp.maximum(cnt_ref[...], 1.0)
    mx = rmax1[...] + rmax2[...] + rmax3[...]
    mn = (rsum1[...] + rsum2[...] + rsum3[...]) / cnt
    xg = jnp.concatenate([mx, mn], axis=1)
    xg = jnp.dot(xg, wg1_ref[...], preferred_element_type=jnp.float32)
    xg = jnp.maximum(_bn64(xg + bg1_ref[...], g4_ref[...], be4_ref[...]), 0.0)

    xt = jnp.dot(convf_ref[...], wfx_ref[...],
                 preferred_element_type=jnp.float32) + bfx_ref[...]
    xt = _bn64(jnp.maximum(xt, 0.0), g6_ref[...], be6_ref[...])

    xc = jnp.concatenate([xg, xt], axis=1)
    xc = jnp.dot(xc, wf1_ref[...], preferred_element_type=jnp.float32)
    xc = _bn64(jnp.maximum(xc + bf1_ref[...], 0.0), g7_ref[...], be7_ref[...])
    xc = jnp.dot(xc, wf2_ref[...], preferred_element_type=jnp.float32)
    xc = _bn64(jnp.maximum(xc + bf2_ref[...], 0.0), g8_ref[...], be8_ref[...])
    out_ref[...] = jnp.dot(xc, wo_ref[...],
                           preferred_element_type=jnp.float32)[:, 0:1] \
        + bo_ref[...]


def _head(rm1, rm2, rm3, rs1, rs2, rs3, cnt, convf, wfx_pad, p):
    return pl.pallas_call(
        _head_body,
        out_shape=jax.ShapeDtypeStruct((G, 1), jnp.float32),
    )(rm1, rm2, rm3, rs1, rs2, rs3, cnt, convf, wfx_pad,
      p['bfx'].reshape(1, D), p['g6'].reshape(1, D), p['be6'].reshape(1, D),
      p['Wg1'], p['bg1'].reshape(1, D), p['g4'].reshape(1, D),
      p['be4'].reshape(1, D),
      p['Wf1'], p['bf1'].reshape(1, 1024), p['g7'].reshape(1, 1024),
      p['be7'].reshape(1, 1024),
      p['Wf2'], p['bf2'].reshape(1, 512), p['g8'].reshape(1, 512),
      p['be8'].reshape(1, 512),
      jnp.pad(p['Wo'], ((0, 0), (0, 127))), p['bo'].reshape(1, 1))


def kernel(x, edge_index, batch, target, params):
    p = params
    xp = jnp.pad(x, ((0, NP - N), (0, 0)))
    src = edge_index[0]
    dst = edge_index[1]
    ar = jnp.arange(EP - E, dtype=jnp.int32)
    src_pad = jnp.concatenate([src, (ar * 13) % N])
    dst_pad = jnp.concatenate([dst, N + ar % (NP - N)])
    dst2 = dst_pad.reshape(EP // WIN, WIN)
    batch_pad = jnp.pad(batch, (0, NP - N), constant_values=G)
    bat2 = batch_pad.reshape(NBLK, 1, BLK)
    gmin = bat2[:, 0, 0]
    gmax = bat2[:, 0, -1]
    zero2d = jnp.zeros((NP, D), jnp.float32)
    zero1d = jnp.zeros((NP,), jnp.float32)

    degp = _sc_deg(dst2, zero1d)
    dinv, y = _pre(degp, xp, p['W1'])

    rms, rss = [], []
    cnt = None
    for li, l in enumerate(['1', '2', '3']):
        parts = _sc_feat_agg(y, src_pad, dst2, zero2d)
        t, stats = _p1(parts, y, dinv, p['bconv' + l])
        ssq = _p1b(t, stats, dinv)
        h, z = _p2(t, stats, ssq, p['g' + l], p['be' + l],
                   jnp.pad(p['Wp' + l], ((0, 0), (0, D - 1))), dinv)
        saggp = _sc_scal_agg(z.reshape(NP), src_pad, dst2, zero1d)
        w_next = p['W' + str(li + 2)] if li < 2 else None
        outs = _p3(gmin, gmax, h, z, saggp, dinv, p['bp' + l], bat2, w_next)
        rms.append(outs[0])
        rss.append(outs[1])
        cnt = outs[2]
        if li < 2:
            y = outs[3]

    wt = jnp.transpose(p['Wc'], (2, 0, 1)).reshape(8 * NF, 640)
    conv = _conv(wt, target, p['bc'].reshape(NF, 1))
    convf = conv.reshape(G, NF * LPROT)
    wfx_pad = jnp.pad(p['Wfx'].reshape(NF, LCONV, D),
                      ((0, 0), (0, LPROT - LCONV), (0, 0))
                      ).reshape(NF * LPROT, D)
    return _head(rms[0], rms[1], rms[2], rss[0], rss[1], rss[2], cnt,
                 convf, wfx_pad, p)


# speedup vs baseline: 20.7102x; 1.1751x over previous
import functools

import jax
import jax.numpy as jnp
from jax import lax
from jax.experimental import pallas as pl
from jax.experimental.pallas import tpu as pltpu
from jax.experimental.pallas import tpu_sc as plsc

N = 10000
NP = 10240
E = 320000
WIN = 128
EW = 80
NWK = 32
EPW = EW * WIN
EP = NWK * EPW
G = 64
D = 128
BLK = 512
NBLK = NP // BLK
NF = 32
LPROT = 1024
LCONV = LPROT - 8 + 1
ROWS_PER_SUB = NP // 16

_SC_MESH = dict(core_axis_name="c", subcore_axis_name="s", num_cores=2,
                num_subcores=16)


def _sc_feat_body(y_hbm, src_hbm, dst_hbm, zero_hbm, out_hbm,
                  sb0, sb1, db0, db1, r0, r1, acc_sh,
                  g0, g1, s0, s1):
    rows = (r0, r1)
    srcb = (sb0, sb1)
    dstb = (db0, db1)
    gsem = (g0, g1)
    ssem = (s0, s1)
    cid = lax.axis_index("c")
    sid = lax.axis_index("s")
    wid = cid * 16 + sid
    base = wid * EPW
    pltpu.sync_copy(zero_hbm.at[pl.ds(sid * ROWS_PER_SUB, ROWS_PER_SUB)],
                    acc_sh.at[pl.ds(sid * ROWS_PER_SUB, ROWS_PER_SUB)])
    plsc.subcore_barrier()

    def stage_idx(w, b):
        pltpu.sync_copy(src_hbm.at[pl.ds(base + w * WIN, WIN)], srcb[b])
        pltpu.sync_copy(dst_hbm.at[pl.ds(base + w * WIN, WIN)], dstb[b])

    def fire_gather(w, b):
        pltpu.async_copy(y_hbm.at[srcb[b]], rows[b], gsem[b])

    def drain_gather(b):
        pltpu.make_async_copy(y_hbm.at[pl.ds(0, WIN)], rows[b],
                              gsem[b]).wait()

    def fire_scatter(b):
        pltpu.async_copy(rows[b], acc_sh.at[dstb[b]], ssem[b], add=True)

    def drain_scatter(b):
        pltpu.make_async_copy(rows[b], acc_sh.at[dstb[b]], ssem[b]).wait()

    stage_idx(0, 0)
    fire_gather(0, 0)

    @pl.loop(0, EW, step=2)
    def _pair(j):
        for b in range(2):
            w = j + b
            o = 1 - b
            wn = w + 1

            @pl.when(wn < EW)
            def _():
                @pl.when(wn >= 2)
                def _():
                    drain_scatter(o)
                stage_idx(wn, o)
                fire_gather(wn, o)

            drain_gather(b)
            fire_scatter(b)

    drain_scatter(0)
    drain_scatter(1)
    plsc.subcore_barrier()
    pltpu.sync_copy(acc_sh.at[pl.ds(sid * ROWS_PER_SUB, ROWS_PER_SUB)],
                    out_hbm.at[cid, pl.ds(sid * ROWS_PER_SUB, ROWS_PER_SUB)])


def _sc_feat_agg(y, src_flat, dst_flat, zero2d):
    return pl.kernel(
        _sc_feat_body,
        out_type=jax.ShapeDtypeStruct((2, NP, D), jnp.float32),
        mesh=plsc.VectorSubcoreMesh(**_SC_MESH),
        scratch_types=[
            pltpu.VMEM((WIN,), jnp.int32),
            pltpu.VMEM((WIN,), jnp.int32),
            pltpu.VMEM((WIN,), jnp.int32),
            pltpu.VMEM((WIN,), jnp.int32),
            pltpu.VMEM((WIN, D), jnp.float32),
            pltpu.VMEM((WIN, D), jnp.float32),
            pltpu.VMEM_SHARED((NP, D), jnp.float32),
            pltpu.SemaphoreType.DMA,
            pltpu.SemaphoreType.DMA,
            pltpu.SemaphoreType.DMA,
            pltpu.SemaphoreType.DMA,
        ],
    )(y, src_flat, dst_flat, zero2d)


def _sc_scal_body(z_hbm, src_hbm, dst2_hbm, zero_hbm, out_hbm,
                  src_v, dstw_v, vals_v, acc_sh, gsem, ssem):
    cid = lax.axis_index("c")
    sid = lax.axis_index("s")
    wid = cid * 16 + sid
    pltpu.sync_copy(zero_hbm.at[pl.ds(sid * ROWS_PER_SUB, ROWS_PER_SUB)],
                    acc_sh.at[pl.ds(sid * ROWS_PER_SUB, ROWS_PER_SUB)])
    pltpu.sync_copy(src_hbm.at[pl.ds(wid * EPW, EPW)], src_v)
    pltpu.sync_copy(dst2_hbm.at[pl.ds(wid * EW, EW)], dstw_v)
    plsc.subcore_barrier()

    pltpu.async_copy(z_hbm.at[src_v], vals_v, gsem).wait()

    @pl.loop(0, EW)
    def _fire(j):
        pltpu.async_copy(vals_v.at[pl.ds(j * WIN, WIN)],
                         acc_sh.at[dstw_v.at[j]], ssem, add=True)

    @pl.loop(0, EW)
    def _drain(j):
        pltpu.make_async_copy(vals_v.at[pl.ds(0, WIN)],
                              acc_sh.at[dstw_v.at[0]], ssem).wait()

    plsc.subcore_barrier()
    pltpu.sync_copy(acc_sh.at[pl.ds(sid * ROWS_PER_SUB, ROWS_PER_SUB)],
                    out_hbm.at[cid, pl.ds(sid * ROWS_PER_SUB, ROWS_PER_SUB)])


def _sc_scal_agg(z, src_flat, dst2, zero1d):
    return pl.kernel(
        _sc_scal_body,
        out_type=jax.ShapeDtypeStruct((2, NP), jnp.float32),
        mesh=plsc.VectorSubcoreMesh(**_SC_MESH),
        scratch_types=[
            pltpu.VMEM((EPW,), jnp.int32),
            pltpu.VMEM((EW, WIN), jnp.int32),
            pltpu.VMEM((EPW,), jnp.float32),
            pltpu.VMEM_SHARED((NP,), jnp.float32),
            pltpu.SemaphoreType.DMA,
            pltpu.SemaphoreType.DMA,
        ],
    )(z, src_flat, dst2, zero1d)


def _sc_deg_body(dst2_hbm, zero_hbm, out_hbm, dstw_v, vals_v, acc_sh, ssem):
    cid = lax.axis_index("c")
    sid = lax.axis_index("s")
    wid = cid * 16 + sid
    pltpu.sync_copy(zero_hbm.at[pl.ds(sid * ROWS_PER_SUB, ROWS_PER_SUB)],
                    acc_sh.at[pl.ds(sid * ROWS_PER_SUB, ROWS_PER_SUB)])
    for i in range(WIN // 16):
        vals_v[pl.ds(i * 16, 16)] = jnp.ones((16,), jnp.float32)
    pltpu.sync_copy(dst2_hbm.at[pl.ds(wid * EW, EW)], dstw_v)
    plsc.subcore_barrier()

    @pl.loop(0, EW)
    def _fire(j):
        pltpu.async_copy(vals_v, acc_sh.at[dstw_v.at[j]], ssem, add=True)

    @pl.loop(0, EW)
    def _drain(j):
        pltpu.make_async_copy(vals_v, acc_sh.at[dstw_v.at[0]], ssem).wait()

    plsc.subcore_barrier()
    pltpu.sync_copy(acc_sh.at[pl.ds(sid * ROWS_PER_SUB, ROWS_PER_SUB)],
                    out_hbm.at[cid, pl.ds(sid * ROWS_PER_SUB, ROWS_PER_SUB)])


def _sc_deg(dst2, zero1d):
    return pl.kernel(
        _sc_deg_body,
        out_type=jax.ShapeDtypeStruct((2, NP), jnp.float32),
        mesh=plsc.VectorSubcoreMesh(**_SC_MESH),
        scratch_types=[
            pltpu.VMEM((EW, WIN), jnp.int32),
            pltpu.VMEM((WIN,), jnp.float32),
            pltpu.VMEM_SHARED((NP,), jnp.float32),
            pltpu.SemaphoreType.DMA,
        ],
    )(dst2, zero1d)


def _pre_body(degp_ref, x_ref, w_ref, dinv_ref, y_ref):
    i = pl.program_id(0)
    degrow = degp_ref[0:1, :] + degp_ref[1:2, :] + 1.0
    deg = jnp.transpose(degrow)
    rowid = lax.broadcasted_iota(jnp.int32, (BLK, 1), 0) + i * BLK
    dinv = jnp.where(rowid < N, lax.rsqrt(deg), 0.0)
    dinv_ref[...] = dinv
    y_ref[...] = dinv * jnp.dot(x_ref[...], w_ref[...],
                                preferred_element_type=jnp.float32)


def _pre(degp, xp, w):
    return pl.pallas_call(
        _pre_body,
        grid=(NBLK,),
        in_specs=[
            pl.BlockSpec((2, BLK), lambda i: (0, i)),
            pl.BlockSpec((BLK, D), lambda i: (i, 0)),
            pl.BlockSpec((D, D), lambda i: (0, 0)),
        ],
        out_specs=[
            pl.BlockSpec((BLK, 1), lambda i: (i, 0)),
            pl.BlockSpec((BLK, D), lambda i: (i, 0)),
        ],
        out_shape=[
            jax.ShapeDtypeStruct((NP, 1), jnp.float32),
            jax.ShapeDtypeStruct((NP, D), jnp.float32),
        ],
    )(degp, xp, w)


def _p1_body(parts_ref, y_ref, dinv_ref, b_ref, t_ref, stats_ref):
    i = pl.program_id(0)
    dinv = dinv_ref[...]
    t = dinv * (parts_ref[0] + parts_ref[1] + y_ref[...]) + b_ref[...]
    t_ref[...] = t
    mask = (dinv > 0.0).astype(jnp.float32)
    tm = t * mask

    @pl.when(i == 0)
    def _():
        stats_ref[...] = jnp.zeros_like(stats_ref)

    stats_ref[0:1, :] += jnp.sum(tm, axis=0, keepdims=True)


def _p1(parts, y, dinv, b):
    return pl.pallas_call(
        _p1_body,
        grid=(NBLK,),
        in_specs=[
            pl.BlockSpec((2, BLK, D), lambda i: (0, i, 0)),
            pl.BlockSpec((BLK, D), lambda i: (i, 0)),
            pl.BlockSpec((BLK, 1), lambda i: (i, 0)),
            pl.BlockSpec((1, D), lambda i: (0, 0)),
        ],
        out_specs=[
            pl.BlockSpec((BLK, D), lambda i: (i, 0)),
            pl.BlockSpec((8, D), lambda i: (0, 0)),
        ],
        out_shape=[
            jax.ShapeDtypeStruct((NP, D), jnp.float32),
            jax.ShapeDtypeStruct((8, D), jnp.float32),
        ],
    )(parts, y, dinv, b.reshape(1, D))


def _p1b_body(t_ref, stats_ref, dinv_ref, ssq_ref):
    i = pl.program_id(0)
    mean = stats_ref[0:1, :] * (1.0 / N)
    mask = (dinv_ref[...] > 0.0).astype(jnp.float32)
    dv = (t_ref[...] - mean) * mask

    @pl.when(i == 0)
    def _():
        ssq_ref[...] = jnp.zeros_like(ssq_ref)

    ssq_ref[0:1, :] += jnp.sum(dv * dv, axis=0, keepdims=True)


def _p1b(t, stats, dinv):
    return pl.pallas_call(
        _p1b_body,
        grid=(NBLK,),
        in_specs=[
            pl.BlockSpec((BLK, D), lambda i: (i, 0)),
            pl.BlockSpec((8, D), lambda i: (0, 0)),
            pl.BlockSpec((BLK, 1), lambda i: (i, 0)),
        ],
        out_specs=pl.BlockSpec((8, D), lambda i: (0, 0)),
        out_shape=jax.ShapeDtypeStruct((8, D), jnp.float32),
    )(t, stats, dinv)


def _p2_body(t_ref, stats_ref, ssq_ref, g_ref, be_ref, wp_ref, dinv_ref,
             h_ref, z_ref):
    mean = stats_ref[0:1, :] * (1.0 / N)
    var = ssq_ref[0:1, :] * (1.0 / N)
    h = jnp.maximum(
        (t_ref[...] - mean) / jnp.sqrt(var + 1e-5) * g_ref[...] + be_ref[...],
        0.0)
    h_ref[...] = h
    z = jnp.dot(h, wp_ref[...], preferred_element_type=jnp.float32)[:, 0:1]
    z_ref[...] = dinv_ref[...] * z


def _p2(t, stats, ssq, g, be, wp_row, dinv):
    return pl.pallas_call(
        _p2_body,
        grid=(NBLK,),
        in_specs=[
            pl.BlockSpec((BLK, D), lambda i: (i, 0)),
            pl.BlockSpec((8, D), lambda i: (0, 0)),
            pl.BlockSpec((8, D), lambda i: (0, 0)),
            pl.BlockSpec((1, D), lambda i: (0, 0)),
            pl.BlockSpec((1, D), lambda i: (0, 0)),
            pl.BlockSpec((D, D), lambda i: (0, 0)),
            pl.BlockSpec((BLK, 1), lambda i: (i, 0)),
        ],
        out_specs=[
            pl.BlockSpec((BLK, D), lambda i: (i, 0)),
            pl.BlockSpec((BLK, 1), lambda i: (i, 0)),
        ],
        out_shape=[
            jax.ShapeDtypeStruct((NP, D), jnp.float32),
            jax.ShapeDtypeStruct((NP, 1), jnp.float32),
        ],
    )(t, stats, ssq, g.reshape(1, D), be.reshape(1, D), wp_row, dinv)


def _p3_body(gmin_ref, gmax_ref, h_ref, z_ref, saggp_ref, dinv_ref, bp_ref,
             batch_ref, w_ref, rmax_ref, rsum_ref, cnt_ref, y_ref=None,
             *, with_y):
    i = pl.program_id(0)
    sagg = jnp.transpose(saggp_ref[0:1, :] + saggp_ref[1:2, :])
    s = dinv_ref[...] * (sagg + z_ref[...]) + bp_ref[...]
    gt = h_ref[...] * jnp.tanh(s)

    bvec = batch_ref[0]
    gids = lax.broadcasted_iota(jnp.int32, (G, 1), 0)
    onehot = (bvec == gids).astype(jnp.float32)

    @pl.when(i == 0)
    def _():
        rsum_ref[...] = jnp.zeros_like(rsum_ref)
        cnt_ref[...] = jnp.zeros_like(cnt_ref)
        rmax_ref[...] = jnp.full_like(rmax_ref, -1e30)

    rsum_ref[...] += jnp.dot(onehot, gt, preferred_element_type=jnp.float32,
                             precision=lax.Precision.HIGHEST)
    cnt_ref[...] += jnp.sum(onehot, axis=1, keepdims=True)

    bcol = jnp.transpose(bvec)
    gmin = gmin_ref[i]
    gmax = gmax_ref[i]
    for gg in range(G):
        @pl.when((gmin <= gg) & (gg <= gmax))
        def _():
            m = bcol == gg
            colmax = jnp.max(jnp.where(m, gt, -1e30), axis=0, keepdims=True)
            rmax_ref[gg:gg + 1, :] = jnp.maximum(rmax_ref[gg:gg + 1, :],
                                                 colmax)
    if with_y:
        y_ref[...] = dinv_ref[...] * jnp.dot(
            gt, w_ref[...], preferred_element_type=jnp.float32)


def _p3(gmin, gmax, h, z, saggp, dinv, bp, bat2, w_next):
    with_y = w_next is not None
    if not with_y:
        w_next = jnp.zeros((D, D), jnp.float32)
    out_specs = [
        pl.BlockSpec((G, D), lambda i: (0, 0)),
        pl.BlockSpec((G, D), lambda i: (0, 0)),
        pl.BlockSpec((G, 1), lambda i: (0, 0)),
    ]
    out_shape = [
        jax.ShapeDtypeStruct((G, D), jnp.float32),
        jax.ShapeDtypeStruct((G, D), jnp.float32),
        jax.ShapeDtypeStruct((G, 1), jnp.float32),
    ]
    if with_y:
        out_specs.append(pl.BlockSpec((BLK, D), lambda i: (i, 0)))
        out_shape.append(jax.ShapeDtypeStruct((NP, D), jnp.float32))
    return pl.pallas_call(
        functools.partial(_p3_body, with_y=with_y),
        grid=(NBLK,),
        in_specs=[
            pl.BlockSpec(memory_space=pltpu.SMEM),
            pl.BlockSpec(memory_space=pltpu.SMEM),
            pl.BlockSpec((BLK, D), lambda i: (i, 0)),
            pl.BlockSpec((BLK, 1), lambda i: (i, 0)),
            pl.BlockSpec((2, BLK), lambda i: (0, i)),
            pl.BlockSpec((BLK, 1), lambda i: (i, 0)),
            pl.BlockSpec((1, 1), lambda i: (0, 0)),
            pl.BlockSpec((1, 1, BLK), lambda i: (i, 0, 0)),
            pl.BlockSpec((D, D), lambda i: (0, 0)),
        ],
        out_specs=out_specs,
        out_shape=out_shape,
    )(gmin, gmax, h, z, saggp, dinv, bp.reshape(1, 1), bat2, w_next)


def _conv_body(wt_ref, tgt_ref, bc_ref, out_ref):
    p = jnp.dot(wt_ref[...], tgt_ref[0], preferred_element_type=jnp.float32)
    acc = bc_ref[...] + jnp.zeros((NF, LPROT), jnp.float32)
    acc = acc + p[0:NF, :]
    for k in range(1, 8):
        acc = acc + jnp.roll(p[k * NF:(k + 1) * NF, :], -k, axis=1)
    out_ref[0] = acc


def _conv(wt, target, bc):
    return pl.pallas_call(
        _conv_body,
        grid=(G,),
        in_specs=[
            pl.BlockSpec((8 * NF, 640), lambda b: (0, 0)),
            pl.BlockSpec((1, 640, LPROT), lambda b: (b, 0, 0)),
            pl.BlockSpec((NF, 1), lambda b: (0, 0)),
        ],
        out_specs=pl.BlockSpec((1, NF, LPROT), lambda b: (b, 0, 0)),
        out_shape=jax.ShapeDtypeStruct((G, NF, LPROT), jnp.float32),
    )(wt, target, bc)


def _bn64(v, g, be):
    mean = jnp.mean(v, axis=0, keepdims=True)
    dv = v - mean
    var = jnp.mean(dv * dv, axis=0, keepdims=True)
    return dv / jnp.sqrt(var + 1e-5) * g + be


def _head_body(rmax1, rmax2, rmax3, rsum1, rsum2, rsum3, cnt_ref,
               convf_ref, wfx_ref, bfx_ref, g6_ref, be6_ref,
               wg1_ref, bg1_ref, g4_ref, be4_ref,
               wf1_ref, bf1_ref, g7_ref, be7_ref,
               wf2_ref, bf2_ref, g8_ref, be8_ref,
               wo_ref, bo_ref, out_ref):
    cnt = jnp.maximum(cnt_ref[...], 1.0)
    mx = rmax1[...] + rmax2[...] + rmax3[...]
    mn = (rsum1[...] + rsum2[...] + rsum3[...]) / cnt
    xg = jnp.concatenate([mx, mn], axis=1)
    xg = jnp.dot(xg, wg1_ref[...], preferred_element_type=jnp.float32)
    xg = jnp.maximum(_bn64(xg + bg1_ref[...], g4_ref[...], be4_ref[...]), 0.0)

    xt = jnp.dot(convf_ref[...], wfx_ref[...],
                 preferred_element_type=jnp.float32) + bfx_ref[...]
    xt = _bn64(jnp.maximum(xt, 0.0), g6_ref[...], be6_ref[...])

    xc = jnp.concatenate([xg, xt], axis=1)
    xc = jnp.dot(xc, wf1_ref[...], preferred_element_type=jnp.float32)
    xc = _bn64(jnp.maximum(xc + bf1_ref[...], 0.0), g7_ref[...], be7_ref[...])
    xc = jnp.dot(xc, wf2_ref[...], preferred_element_type=jnp.float32)
    xc = _bn64(jnp.maximum(xc + bf2_ref[...], 0.0), g8_ref[...], be8_ref[...])
    out_ref[...] = jnp.dot(xc, wo_ref[...],
                           preferred_element_type=jnp.float32)[:, 0:1] \
        + bo_ref[...]


def _head(rm1, rm2, rm3, rs1, rs2, rs3, cnt, convf, wfx_pad, p):
    return pl.pallas_call(
        _head_body,
        out_shape=jax.ShapeDtypeStruct((G, 1), jnp.float32),
    )(rm1, rm2, rm3, rs1, rs2, rs3, cnt, convf, wfx_pad,
      p['bfx'].reshape(1, D), p['g6'].reshape(1, D), p['be6'].reshape(1, D),
      p['Wg1'], p['bg1'].reshape(1, D), p['g4'].reshape(1, D),
      p['be4'].reshape(1, D),
      p['Wf1'], p['bf1'].reshape(1, 1024), p['g7'].reshape(1, 1024),
      p['be7'].reshape(1, 1024),
      p['Wf2'], p['bf2'].reshape(1, 512), p['g8'].reshape(1, 512),
      p['be8'].reshape(1, 512),
      jnp.pad(p['Wo'], ((0, 0), (0, 127))), p['bo'].reshape(1, 1))


def kernel(x, edge_index, batch, target, params):
    p = params
    xp = jnp.pad(x, ((0, NP - N), (0, 0)))
    src = edge_index[0]
    dst = edge_index[1]
    ar = jnp.arange(EP - E, dtype=jnp.int32)
    src_pad = jnp.concatenate([src, (ar * 13) % N])
    dst_pad = jnp.concatenate([dst, N + ar % (NP - N)])
    dst2 = dst_pad.reshape(EP // WIN, WIN)
    batch_pad = jnp.pad(batch, (0, NP - N), constant_values=G)
    bat2 = batch_pad.reshape(NBLK, 1, BLK)
    gmin = bat2[:, 0, 0]
    gmax = bat2[:, 0, -1]
    zero2d = jnp.zeros((NP, D), jnp.float32)
    zero1d = jnp.zeros((NP,), jnp.float32)

    degp = _sc_deg(dst2, zero1d)
    dinv, y = _pre(degp, xp, p['W1'])

    rms, rss = [], []
    cnt = None
    for li, l in enumerate(['1', '2', '3']):
        parts = _sc_feat_agg(y, src_pad, dst_pad, zero2d)
        t, stats = _p1(parts, y, dinv, p['bconv' + l])
        ssq = _p1b(t, stats, dinv)
        h, z = _p2(t, stats, ssq, p['g' + l], p['be' + l],
                   jnp.pad(p['Wp' + l], ((0, 0), (0, D - 1))), dinv)
        saggp = _sc_scal_agg(z.reshape(NP), src_pad, dst2, zero1d)
        w_next = p['W' + str(li + 2)] if li < 2 else None
        outs = _p3(gmin, gmax, h, z, saggp, dinv, p['bp' + l], bat2, w_next)
        rms.append(outs[0])
        rss.append(outs[1])
        cnt = outs[2]
        if li < 2:
            y = outs[3]

    wt = jnp.transpose(p['Wc'], (2, 0, 1)).reshape(8 * NF, 640)
    conv = _conv(wt, target, p['bc'].reshape(NF, 1))
    convf = conv.reshape(G, NF * LPROT)
    wfx_pad = jnp.pad(p['Wfx'].reshape(NF, LCONV, D),
                      ((0, 0), (0, LPROT - LCONV), (0, 0))
                      ).reshape(NF * LPROT, D)
    return _head(rms[0], rms[1], rms[2], rss[0], rss[1], rss[2], cnt,
                 convf, wfx_pad, p)


# speedup vs baseline: 23.7157x; 1.1451x over previous
import functools

import jax
import jax.numpy as jnp
from jax import lax
from jax.experimental import pallas as pl
from jax.experimental.pallas import tpu as pltpu
from jax.experimental.pallas import tpu_sc as plsc

N = 10000
NP = 10240
E = 320000
WIN = 128
EW = 80
NWK = 32
EPW = EW * WIN
EP = NWK * EPW
G = 64
D = 128
BLK = 512
NBLK = NP // BLK
NF = 32
LPROT = 1024
LCONV = LPROT - 8 + 1
ROWS_PER_SUB = NP // 16

_SC_MESH = dict(core_axis_name="c", subcore_axis_name="s", num_cores=2,
                num_subcores=16)


def _sc_feat_body(y_hbm, src2_hbm, dst_hbm, zero_hbm, out_hbm,
                  src_v, db0, db1, r0, r1, acc_sh,
                  g0, g1, s0, s1, i0, i1):
    rows = (r0, r1)
    dstb = (db0, db1)
    gsem = (g0, g1)
    ssem = (s0, s1)
    isem = (i0, i1)
    cid = lax.axis_index("c")
    sid = lax.axis_index("s")
    wid = cid * 16 + sid
    base = wid * EPW
    pltpu.sync_copy(zero_hbm.at[pl.ds(sid * ROWS_PER_SUB, ROWS_PER_SUB)],
                    acc_sh.at[pl.ds(sid * ROWS_PER_SUB, ROWS_PER_SUB)])
    pltpu.sync_copy(src2_hbm.at[pl.ds(wid * EW, EW)], src_v)
    plsc.subcore_barrier()

    def stage_dst(w, b):
        pltpu.async_copy(dst_hbm.at[pl.ds(base + w * WIN, WIN)], dstb[b],
                         isem[b])

    def drain_dst(b):
        pltpu.make_async_copy(dst_hbm.at[pl.ds(0, WIN)], dstb[b],
                              isem[b]).wait()

    def fire_gather(w, b):
        pltpu.async_copy(y_hbm.at[src_v.at[w]], rows[b], gsem[b])

    def drain_gather(b):
        pltpu.make_async_copy(y_hbm.at[pl.ds(0, WIN)], rows[b],
                              gsem[b]).wait()

    def fire_scatter(b):
        pltpu.async_copy(rows[b], acc_sh.at[dstb[b]], ssem[b], add=True)

    def drain_scatter(b):
        pltpu.make_async_copy(rows[b], acc_sh.at[dstb[b]], ssem[b]).wait()

    stage_dst(0, 0)
    fire_gather(0, 0)

    @pl.loop(0, EW, step=2)
    def _pair(j):
        for b in range(2):
            w = j + b
            o = 1 - b
            wn = w + 1

            @pl.when(wn < EW)
            def _():
                @pl.when(wn >= 2)
                def _():
                    drain_scatter(o)
                stage_dst(wn, o)
                fire_gather(wn, o)

            drain_gather(b)
            drain_dst(b)
            fire_scatter(b)

    drain_scatter(0)
    drain_scatter(1)
    plsc.subcore_barrier()
    pltpu.sync_copy(acc_sh.at[pl.ds(sid * ROWS_PER_SUB, ROWS_PER_SUB)],
                    out_hbm.at[cid, pl.ds(sid * ROWS_PER_SUB, ROWS_PER_SUB)])


def _sc_feat_agg(y, src2, dst_flat, zero2d):
    return pl.kernel(
        _sc_feat_body,
        out_type=jax.ShapeDtypeStruct((2, NP, D), jnp.float32),
        mesh=plsc.VectorSubcoreMesh(**_SC_MESH),
        scratch_types=[
            pltpu.VMEM((EW, WIN), jnp.int32),
            pltpu.VMEM((WIN,), jnp.int32),
            pltpu.VMEM((WIN,), jnp.int32),
            pltpu.VMEM((WIN, D), jnp.float32),
            pltpu.VMEM((WIN, D), jnp.float32),
            pltpu.VMEM_SHARED((NP, D), jnp.float32),
            pltpu.SemaphoreType.DMA,
            pltpu.SemaphoreType.DMA,
            pltpu.SemaphoreType.DMA,
            pltpu.SemaphoreType.DMA,
            pltpu.SemaphoreType.DMA,
            pltpu.SemaphoreType.DMA,
        ],
    )(y, src2, dst_flat, zero2d)


def _sc_scal_body(z_hbm, src_hbm, dst2_hbm, zero_hbm, out_hbm,
                  src_v, dstw_v, vals_v, acc_sh, gsem, ssem):
    cid = lax.axis_index("c")
    sid = lax.axis_index("s")
    wid = cid * 16 + sid
    pltpu.sync_copy(zero_hbm.at[pl.ds(sid * ROWS_PER_SUB, ROWS_PER_SUB)],
                    acc_sh.at[pl.ds(sid * ROWS_PER_SUB, ROWS_PER_SUB)])
    pltpu.sync_copy(src_hbm.at[pl.ds(wid * EPW, EPW)], src_v)
    pltpu.sync_copy(dst2_hbm.at[pl.ds(wid * EW, EW)], dstw_v)
    plsc.subcore_barrier()

    pltpu.async_copy(z_hbm.at[src_v], vals_v, gsem).wait()

    @pl.loop(0, EW)
    def _fire(j):
        pltpu.async_copy(vals_v.at[pl.ds(j * WIN, WIN)],
                         acc_sh.at[dstw_v.at[j]], ssem, add=True)

    @pl.loop(0, EW)
    def _drain(j):
        pltpu.make_async_copy(vals_v.at[pl.ds(0, WIN)],
                              acc_sh.at[dstw_v.at[0]], ssem).wait()

    plsc.subcore_barrier()
    pltpu.sync_copy(acc_sh.at[pl.ds(sid * ROWS_PER_SUB, ROWS_PER_SUB)],
                    out_hbm.at[cid, pl.ds(sid * ROWS_PER_SUB, ROWS_PER_SUB)])


def _sc_scal_agg(z, src_flat, dst2, zero1d):
    return pl.kernel(
        _sc_scal_body,
        out_type=jax.ShapeDtypeStruct((2, NP), jnp.float32),
        mesh=plsc.VectorSubcoreMesh(**_SC_MESH),
        scratch_types=[
            pltpu.VMEM((EPW,), jnp.int32),
            pltpu.VMEM((EW, WIN), jnp.int32),
            pltpu.VMEM((EPW,), jnp.float32),
            pltpu.VMEM_SHARED((NP,), jnp.float32),
            pltpu.SemaphoreType.DMA,
            pltpu.SemaphoreType.DMA,
        ],
    )(z, src_flat, dst2, zero1d)


def _sc_deg_body(dst2_hbm, zero_hbm, out_hbm, dstw_v, vals_v, acc_sh, ssem):
    cid = lax.axis_index("c")
    sid = lax.axis_index("s")
    wid = cid * 16 + sid
    pltpu.sync_copy(zero_hbm.at[pl.ds(sid * ROWS_PER_SUB, ROWS_PER_SUB)],
                    acc_sh.at[pl.ds(sid * ROWS_PER_SUB, ROWS_PER_SUB)])
    for i in range(WIN // 16):
        vals_v[pl.ds(i * 16, 16)] = jnp.ones((16,), jnp.float32)
    pltpu.sync_copy(dst2_hbm.at[pl.ds(wid * EW, EW)], dstw_v)
    plsc.subcore_barrier()

    @pl.loop(0, EW)
    def _fire(j):
        pltpu.async_copy(vals_v, acc_sh.at[dstw_v.at[j]], ssem, add=True)

    @pl.loop(0, EW)
    def _drain(j):
        pltpu.make_async_copy(vals_v, acc_sh.at[dstw_v.at[0]], ssem).wait()

    plsc.subcore_barrier()
    pltpu.sync_copy(acc_sh.at[pl.ds(sid * ROWS_PER_SUB, ROWS_PER_SUB)],
                    out_hbm.at[cid, pl.ds(sid * ROWS_PER_SUB, ROWS_PER_SUB)])


def _sc_deg(dst2, zero1d):
    return pl.kernel(
        _sc_deg_body,
        out_type=jax.ShapeDtypeStruct((2, NP), jnp.float32),
        mesh=plsc.VectorSubcoreMesh(**_SC_MESH),
        scratch_types=[
            pltpu.VMEM((EW, WIN), jnp.int32),
            pltpu.VMEM((WIN,), jnp.float32),
            pltpu.VMEM_SHARED((NP,), jnp.float32),
            pltpu.SemaphoreType.DMA,
        ],
    )(dst2, zero1d)


def _pre_body(degp_ref, x_ref, w_ref, dinv_ref, y_ref):
    i = pl.program_id(0)
    degrow = degp_ref[0:1, :] + degp_ref[1:2, :] + 1.0
    deg = jnp.transpose(degrow)
    rowid = lax.broadcasted_iota(jnp.int32, (BLK, 1), 0) + i * BLK
    dinv = jnp.where(rowid < N, lax.rsqrt(deg), 0.0)
    dinv_ref[...] = dinv
    y_ref[...] = dinv * jnp.dot(x_ref[...], w_ref[...],
                                preferred_element_type=jnp.float32)


def _pre(degp, xp, w):
    return pl.pallas_call(
        _pre_body,
        grid=(NBLK,),
        in_specs=[
            pl.BlockSpec((2, BLK), lambda i: (0, i)),
            pl.BlockSpec((BLK, D), lambda i: (i, 0)),
            pl.BlockSpec((D, D), lambda i: (0, 0)),
        ],
        out_specs=[
            pl.BlockSpec((BLK, 1), lambda i: (i, 0)),
            pl.BlockSpec((BLK, D), lambda i: (i, 0)),
        ],
        out_shape=[
            jax.ShapeDtypeStruct((NP, 1), jnp.float32),
            jax.ShapeDtypeStruct((NP, D), jnp.float32),
        ],
    )(degp, xp, w)


def _p1_body(parts_ref, y_ref, dinv_ref, b_ref, t_ref, stats_ref):
    i = pl.program_id(0)
    dinv = dinv_ref[...]
    t = dinv * (parts_ref[0] + parts_ref[1] + y_ref[...]) + b_ref[...]
    t_ref[...] = t
    mask = (dinv > 0.0).astype(jnp.float32)
    tm = t * mask

    @pl.when(i == 0)
    def _():
        stats_ref[...] = jnp.zeros_like(stats_ref)

    stats_ref[0:1, :] += jnp.sum(tm, axis=0, keepdims=True)


def _p1(parts, y, dinv, b):
    return pl.pallas_call(
        _p1_body,
        grid=(NBLK,),
        in_specs=[
            pl.BlockSpec((2, BLK, D), lambda i: (0, i, 0)),
            pl.BlockSpec((BLK, D), lambda i: (i, 0)),
            pl.BlockSpec((BLK, 1), lambda i: (i, 0)),
            pl.BlockSpec((1, D), lambda i: (0, 0)),
        ],
        out_specs=[
            pl.BlockSpec((BLK, D), lambda i: (i, 0)),
            pl.BlockSpec((8, D), lambda i: (0, 0)),
        ],
        out_shape=[
            jax.ShapeDtypeStruct((NP, D), jnp.float32),
            jax.ShapeDtypeStruct((8, D), jnp.float32),
        ],
    )(parts, y, dinv, b.reshape(1, D))


def _p1b_body(t_ref, stats_ref, dinv_ref, ssq_ref):
    i = pl.program_id(0)
    mean = stats_ref[0:1, :] * (1.0 / N)
    mask = (dinv_ref[...] > 0.0).astype(jnp.float32)
    dv = (t_ref[...] - mean) * mask

    @pl.when(i == 0)
    def _():
        ssq_ref[...] = jnp.zeros_like(ssq_ref)

    ssq_ref[0:1, :] += jnp.sum(dv * dv, axis=0, keepdims=True)


def _p1b(t, stats, dinv):
    return pl.pallas_call(
        _p1b_body,
        grid=(NBLK,),
        in_specs=[
            pl.BlockSpec((BLK, D), lambda i: (i, 0)),
            pl.BlockSpec((8, D), lambda i: (0, 0)),
            pl.BlockSpec((BLK, 1), lambda i: (i, 0)),
        ],
        out_specs=pl.BlockSpec((8, D), lambda i: (0, 0)),
        out_shape=jax.ShapeDtypeStruct((8, D), jnp.float32),
    )(t, stats, dinv)


def _p2_body(t_ref, stats_ref, ssq_ref, g_ref, be_ref, wp_ref, dinv_ref,
             h_ref, z_ref):
    mean = stats_ref[0:1, :] * (1.0 / N)
    var = ssq_ref[0:1, :] * (1.0 / N)
    h = jnp.maximum(
        (t_ref[...] - mean) / jnp.sqrt(var + 1e-5) * g_ref[...] + be_ref[...],
        0.0)
    h_ref[...] = h
    z = jnp.dot(h, wp_ref[...], preferred_element_type=jnp.float32)[:, 0:1]
    z_ref[...] = dinv_ref[...] * z


def _p2(t, stats, ssq, g, be, wp_row, dinv):
    return pl.pallas_call(
        _p2_body,
        grid=(NBLK,),
        in_specs=[
            pl.BlockSpec((BLK, D), lambda i: (i, 0)),
            pl.BlockSpec((8, D), lambda i: (0, 0)),
            pl.BlockSpec((8, D), lambda i: (0, 0)),
            pl.BlockSpec((1, D), lambda i: (0, 0)),
            pl.BlockSpec((1, D), lambda i: (0, 0)),
            pl.BlockSpec((D, D), lambda i: (0, 0)),
            pl.BlockSpec((BLK, 1), lambda i: (i, 0)),
        ],
        out_specs=[
            pl.BlockSpec((BLK, D), lambda i: (i, 0)),
            pl.BlockSpec((BLK, 1), lambda i: (i, 0)),
        ],
        out_shape=[
            jax.ShapeDtypeStruct((NP, D), jnp.float32),
            jax.ShapeDtypeStruct((NP, 1), jnp.float32),
        ],
    )(t, stats, ssq, g.reshape(1, D), be.reshape(1, D), wp_row, dinv)


def _p3_body(gmin_ref, gmax_ref, h_ref, z_ref, saggp_ref, dinv_ref, bp_ref,
             batch_ref, w_ref, rmax_ref, rsum_ref, cnt_ref, y_ref=None,
             *, with_y):
    i = pl.program_id(0)
    sagg = jnp.transpose(saggp_ref[0:1, :] + saggp_ref[1:2, :])
    s = dinv_ref[...] * (sagg + z_ref[...]) + bp_ref[...]
    gt = h_ref[...] * jnp.tanh(s)

    bvec = batch_ref[0]
    gids = lax.broadcasted_iota(jnp.int32, (G, 1), 0)
    onehot = (bvec == gids).astype(jnp.float32)

    @pl.when(i == 0)
    def _():
        rsum_ref[...] = jnp.zeros_like(rsum_ref)
        cnt_ref[...] = jnp.zeros_like(cnt_ref)
        rmax_ref[...] = jnp.full_like(rmax_ref, -1e30)

    rsum_ref[...] += jnp.dot(onehot, gt, preferred_element_type=jnp.float32,
                             precision=lax.Precision.HIGHEST)
    cnt_ref[...] += jnp.sum(onehot, axis=1, keepdims=True)

    bcol = jnp.transpose(bvec)
    gmin = gmin_ref[i]
    gmax = gmax_ref[i]
    for gg in range(G):
        @pl.when((gmin <= gg) & (gg <= gmax))
        def _():
            m = bcol == gg
            colmax = jnp.max(jnp.where(m, gt, -1e30), axis=0, keepdims=True)
            rmax_ref[gg:gg + 1, :] = jnp.maximum(rmax_ref[gg:gg + 1, :],
                                                 colmax)
    if with_y:
        y_ref[...] = dinv_ref[...] * jnp.dot(
            gt, w_ref[...], preferred_element_type=jnp.float32)


def _p3(gmin, gmax, h, z, saggp, dinv, bp, bat2, w_next):
    with_y = w_next is not None
    if not with_y:
        w_next = jnp.zeros((D, D), jnp.float32)
    out_specs = [
        pl.BlockSpec((G, D), lambda i: (0, 0)),
        pl.BlockSpec((G, D), lambda i: (0, 0)),
        pl.BlockSpec((G, 1), lambda i: (0, 0)),
    ]
    out_shape = [
        jax.ShapeDtypeStruct((G, D), jnp.float32),
        jax.ShapeDtypeStruct((G, D), jnp.float32),
        jax.ShapeDtypeStruct((G, 1), jnp.float32),
    ]
    if with_y:
        out_specs.append(pl.BlockSpec((BLK, D), lambda i: (i, 0)))
        out_shape.append(jax.ShapeDtypeStruct((NP, D), jnp.float32))
    return pl.pallas_call(
        functools.partial(_p3_body, with_y=with_y),
        grid=(NBLK,),
        in_specs=[
            pl.BlockSpec(memory_space=pltpu.SMEM),
            pl.BlockSpec(memory_space=pltpu.SMEM),
            pl.BlockSpec((BLK, D), lambda i: (i, 0)),
            pl.BlockSpec((BLK, 1), lambda i: (i, 0)),
            pl.BlockSpec((2, BLK), lambda i: (0, i)),
            pl.BlockSpec((BLK, 1), lambda i: (i, 0)),
            pl.BlockSpec((1, 1), lambda i: (0, 0)),
            pl.BlockSpec((1, 1, BLK), lambda i: (i, 0, 0)),
            pl.BlockSpec((D, D), lambda i: (0, 0)),
        ],
        out_specs=out_specs,
        out_shape=out_shape,
    )(gmin, gmax, h, z, saggp, dinv, bp.reshape(1, 1), bat2, w_next)


def _conv_body(wt_ref, tgt_ref, bc_ref, out_ref):
    p = jnp.dot(wt_ref[...], tgt_ref[0], preferred_element_type=jnp.float32)
    acc = bc_ref[...] + jnp.zeros((NF, LPROT), jnp.float32)
    acc = acc + p[0:NF, :]
    for k in range(1, 8):
        acc = acc + jnp.roll(p[k * NF:(k + 1) * NF, :], -k, axis=1)
    out_ref[0] = acc


def _conv(wt, target, bc):
    return pl.pallas_call(
        _conv_body,
        grid=(G,),
        in_specs=[
            pl.BlockSpec((8 * NF, 640), lambda b: (0, 0)),
            pl.BlockSpec((1, 640, LPROT), lambda b: (b, 0, 0)),
            pl.BlockSpec((NF, 1), lambda b: (0, 0)),
        ],
        out_specs=pl.BlockSpec((1, NF, LPROT), lambda b: (b, 0, 0)),
        out_shape=jax.ShapeDtypeStruct((G, NF, LPROT), jnp.float32),
    )(wt, target, bc)


def _bn64(v, g, be):
    mean = jnp.mean(v, axis=0, keepdims=True)
    dv = v - mean
    var = jnp.mean(dv * dv, axis=0, keepdims=True)
    return dv / jnp.sqrt(var + 1e-5) * g + be


def _head_body(rmax1, rmax2, rmax3, rsum1, rsum2, rsum3, cnt_ref,
               convf_ref, wfx_ref, bfx_ref, g6_ref, be6_ref,
               wg1_ref, bg1_ref, g4_ref, be4_ref,
               wf1_ref, bf1_ref, g7_ref, be7_ref,
               wf2_ref, bf2_ref, g8_ref, be8_ref,
               wo_ref, bo_ref, out_ref):
    cnt = jnp.maximum(cnt_ref[...], 1.0)
    mx = rmax1[...] + rmax2[...] + rmax3[...]
    mn = (rsum1[...] + rsum2[...] + rsum3[...]) / cnt
    xg = jnp.concatenate([mx, mn], axis=1)
    xg = jnp.dot(xg, wg1_ref[...], preferred_element_type=jnp.float32)
    xg = jnp.maximum(_bn64(xg + bg1_ref[...], g4_ref[...], be4_ref[...]), 0.0)

    xt = jnp.dot(convf_ref[...], wfx_ref[...],
                 preferred_element_type=jnp.float32) + bfx_ref[...]
    xt = _bn64(jnp.maximum(xt, 0.0), g6_ref[...], be6_ref[...])

    xc = jnp.concatenate([xg, xt], axis=1)
    xc = jnp.dot(xc, wf1_ref[...], preferred_element_type=jnp.float32)
    xc = _bn64(jnp.maximum(xc + bf1_ref[...], 0.0), g7_ref[...], be7_ref[...])
    xc = jnp.dot(xc, wf2_ref[...], preferred_element_type=jnp.float32)
    xc = _bn64(jnp.maximum(xc + bf2_ref[...], 0.0), g8_ref[...], be8_ref[...])
    out_ref[...] = jnp.dot(xc, wo_ref[...],
                           preferred_element_type=jnp.float32)[:, 0:1] \
        + bo_ref[...]


def _head(rm1, rm2, rm3, rs1, rs2, rs3, cnt, convf, wfx_pad, p):
    return pl.pallas_call(
        _head_body,
        out_shape=jax.ShapeDtypeStruct((G, 1), jnp.float32),
    )(rm1, rm2, rm3, rs1, rs2, rs3, cnt, convf, wfx_pad,
      p['bfx'].reshape(1, D), p['g6'].reshape(1, D), p['be6'].reshape(1, D),
      p['Wg1'], p['bg1'].reshape(1, D), p['g4'].reshape(1, D),
      p['be4'].reshape(1, D),
      p['Wf1'], p['bf1'].reshape(1, 1024), p['g7'].reshape(1, 1024),
      p['be7'].reshape(1, 1024),
      p['Wf2'], p['bf2'].reshape(1, 512), p['g8'].reshape(1, 512),
      p['be8'].reshape(1, 512),
      jnp.pad(p['Wo'], ((0, 0), (0, 127))), p['bo'].reshape(1, 1))


def kernel(x, edge_index, batch, target, params):
    p = params
    xp = jnp.pad(x, ((0, NP - N), (0, 0)))
    src = edge_index[0]
    dst = edge_index[1]
    ar = jnp.arange(EP - E, dtype=jnp.int32)
    src_pad = jnp.concatenate([src, (ar * 13) % N])
    dst_pad = jnp.concatenate([dst, N + ar % (NP - N)])
    dst2 = dst_pad.reshape(EP // WIN, WIN)
    src2 = src_pad.reshape(EP // WIN, WIN)
    batch_pad = jnp.pad(batch, (0, NP - N), constant_values=G)
    bat2 = batch_pad.reshape(NBLK, 1, BLK)
    gmin = bat2[:, 0, 0]
    gmax = bat2[:, 0, -1]
    zero2d = jnp.zeros((NP, D), jnp.float32)
    zero1d = jnp.zeros((NP,), jnp.float32)

    degp = _sc_deg(dst2, zero1d)
    dinv, y = _pre(degp, xp, p['W1'])

    rms, rss = [], []
    cnt = None
    for li, l in enumerate(['1', '2', '3']):
        parts = _sc_feat_agg(y, src2, dst_pad, zero2d)
        t, stats = _p1(parts, y, dinv, p['bconv' + l])
        ssq = _p1b(t, stats, dinv)
        h, z = _p2(t, stats, ssq, p['g' + l], p['be' + l],
                   jnp.pad(p['Wp' + l], ((0, 0), (0, D - 1))), dinv)
        saggp = _sc_scal_agg(z.reshape(NP), src_pad, dst2, zero1d)
        w_next = p['W' + str(li + 2)] if li < 2 else None
        outs = _p3(gmin, gmax, h, z, saggp, dinv, p['bp' + l], bat2, w_next)
        rms.append(outs[0])
        rss.append(outs[1])
        cnt = outs[2]
        if li < 2:
            y = outs[3]

    wt = jnp.transpose(p['Wc'], (2, 0, 1)).reshape(8 * NF, 640)
    conv = _conv(wt, target, p['bc'].reshape(NF, 1))
    convf = conv.reshape(G, NF * LPROT)
    wfx_pad = jnp.pad(p['Wfx'].reshape(NF, LCONV, D),
                      ((0, 0), (0, LPROT - LCONV), (0, 0))
                      ).reshape(NF * LPROT, D)
    return _head(rms[0], rms[1], rms[2], rss[0], rss[1], rss[2], cnt,
                 convf, wfx_pad, p)
